# baseline (speedup 1.0000x reference)
import functools

import jax
import jax.numpy as jnp
from jax import lax
from jax.experimental import pallas as pl
from jax.experimental.pallas import tpu as pltpu
from jax.experimental.pallas import tpu_sc as plsc

_D = 128
_L = 16
_NC = 2
_NS = 16
_NW = _NC * _NS
_C = 80
_NBUF = 5
_RB = 8192


def _sc_permute(x_rows, idx, row0, nrows):
    rows_w = nrows // _NW
    nchunk = rows_w // _C
    ngroups = nchunk // _NBUF

    mesh = plsc.VectorSubcoreMesh(core_axis_name="c", subcore_axis_name="s")

    @functools.partial(
        pl.kernel,
        mesh=mesh,
        out_type=jax.ShapeDtypeStruct((nrows, _D), jnp.float32),
        compiler_params=pltpu.CompilerParams(needs_layout_passes=False),
        scratch_types=[pltpu.VMEM((_D,), jnp.int32)]
        + [pltpu.VMEM((_C, _D), jnp.float32)] * (2 * _NBUF)
        + [pltpu.SemaphoreType.DMA] * (2 * _NBUF),
    )
    def k(x_hbm, idx_hbm, out_hbm, idx_v, *bufs_and_sems):
        in_bufs = bufs_and_sems[0:_NBUF]
        out_bufs = bufs_and_sems[_NBUF:2 * _NBUF]
        sin = bufs_and_sems[2 * _NBUF:3 * _NBUF]
        sout = bufs_and_sems[3 * _NBUF:4 * _NBUF]

        wid = lax.axis_index("s") * _NC + lax.axis_index("c")
        base = wid * rows_w

        pltpu.sync_copy(idx_hbm, idx_v)
        idx_vecs = [idx_v[pl.ds(_L * j, _L)] for j in range(_D // _L)]

        def in_rows(g):
            return pl.ds(row0 + base + g * _C, _C)

        def out_rows(g):
            return pl.ds(base + g * _C, _C)

        for b in range(_NBUF):
            pltpu.async_copy(x_hbm.at[in_rows(b)], in_bufs[b], sin[b])

        def group_body(gi, _):
            g0 = gi * _NBUF
            for b in range(_NBUF):
                g = g0 + b
                in_b = in_bufs[b]
                out_b = out_bufs[b]

                pltpu.make_async_copy(
                    x_hbm.at[in_rows(g)], in_b, sin[b]
                ).wait()

                @pl.when(gi > 0)
                def _wait_out():
                    pltpu.make_async_copy(
                        out_b, out_hbm.at[out_rows(g)], sout[b]
                    ).wait()

                @plsc.parallel_loop(0, _C, unroll=4)
                def row_body(r):
                    row_v = jnp.full((_L,), r, jnp.int32)
                    for j in range(_D // _L):
                        v = plsc.load_gather(in_b, [row_v, idx_vecs[j]])
                        out_b[r, pl.ds(_L * j, _L)] = v

                pltpu.async_copy(out_b, out_hbm.at[out_rows(g)], sout[b])

                @pl.when(g + _NBUF < nchunk)
                def _issue_in():
                    pltpu.async_copy(
                        x_hbm.at[in_rows(g + _NBUF)], in_b, sin[b]
                    )
            return 0

        lax.fori_loop(0, ngroups, group_body, 0)

        for b in range(_NBUF):
            g_last = (ngroups - 1) * _NBUF + b
            pltpu.make_async_copy(
                out_bufs[b], out_hbm.at[out_rows(g_last)], sout[b]
            ).wait()

    return k(x_rows, idx)


def _tc_permute(x_rows, idx, nrows):
    grid = nrows // _RB

    def body(idx_ref, x_ref, o_ref):
        cols = jax.lax.broadcasted_iota(jnp.int32, (_D, _D), 0)
        onehot = (idx_ref[...] == cols).astype(jnp.float32)
        o_ref[...] = jax.lax.dot_general(
            x_ref[...], onehot,
            dimension_numbers=(((1,), (0,)), ((), ())),
            preferred_element_type=jnp.float32,
        )

    return pl.pallas_call(
        body,
        grid=(grid,),
        in_specs=[
            pl.BlockSpec((1, _D), lambda i: (0, 0)),
            pl.BlockSpec((_RB, _D), lambda i: (i, 0)),
        ],
        out_specs=pl.BlockSpec((_RB, _D), lambda i: (i, 0)),
        out_shape=jax.ShapeDtypeStruct((nrows, _D), jnp.float32),
    )(idx.reshape(1, _D), x_rows)


_TC_ROWS = 204800


def kernel(x, indexes):
    b, s, d = x.shape
    n = b * s
    x_rows = x.reshape(n, d)
    idx = indexes.astype(jnp.int32)
    tc_out = _tc_permute(x_rows, idx, _TC_ROWS)
    sc_out = _sc_permute(x_rows, idx, _TC_ROWS, n - _TC_ROWS)
    out = jnp.concatenate([tc_out, sc_out], axis=0)
    return out.reshape(b, s, d)


# speedup vs baseline: 1.1393x; 1.1393x over previous
import functools

import jax
import jax.numpy as jnp
from jax import lax
from jax.experimental import pallas as pl
from jax.experimental.pallas import tpu as pltpu
from jax.experimental.pallas import tpu_sc as plsc

_D = 128
_L = 16
_NC = 2
_NS = 16
_NW = _NC * _NS
_C = 80
_NBUF = 5


def _sc_permute(x_rows, idx):
    n = x_rows.shape[0]
    rows_w = n // _NW
    nchunk = rows_w // _C
    ngroups = nchunk // _NBUF

    mesh = plsc.VectorSubcoreMesh(core_axis_name="c", subcore_axis_name="s")

    @functools.partial(
        pl.kernel,
        mesh=mesh,
        out_type=jax.ShapeDtypeStruct((n, _D), jnp.float32),
        compiler_params=pltpu.CompilerParams(needs_layout_passes=False),
        scratch_types=[pltpu.VMEM((_D,), jnp.int32)]
        + [pltpu.VMEM((_C, _D), jnp.float32)] * (2 * _NBUF)
        + [pltpu.SemaphoreType.DMA] * (2 * _NBUF),
    )
    def k(x_hbm, idx_hbm, out_hbm, idx_v, *bufs_and_sems):
        in_bufs = bufs_and_sems[0:_NBUF]
        out_bufs = bufs_and_sems[_NBUF:2 * _NBUF]
        sin = bufs_and_sems[2 * _NBUF:3 * _NBUF]
        sout = bufs_and_sems[3 * _NBUF:4 * _NBUF]

        wid = lax.axis_index("s") * _NC + lax.axis_index("c")
        base = wid * rows_w

        pltpu.sync_copy(idx_hbm, idx_v)
        idx_vecs = [idx_v[pl.ds(_L * j, _L)] for j in range(_D // _L)]

        def rows(g):
            return pl.ds(base + g * _C, _C)

        for b in range(_NBUF):
            pltpu.async_copy(x_hbm.at[rows(b)], in_bufs[b], sin[b])

        def group_body(gi, _):
            g0 = gi * _NBUF
            for b in range(_NBUF):
                g = g0 + b
                in_b = in_bufs[b]
                out_b = out_bufs[b]

                pltpu.make_async_copy(x_hbm.at[rows(g)], in_b, sin[b]).wait()

                @pl.when(gi > 0)
                def _wait_out():
                    pltpu.make_async_copy(
                        out_b, out_hbm.at[rows(g)], sout[b]
                    ).wait()

                @plsc.parallel_loop(0, _C, unroll=4)
                def row_body(r):
                    row_v = jnp.full((_L,), r, jnp.int32)
                    for j in range(_D // _L):
                        v = plsc.load_gather(in_b, [row_v, idx_vecs[j]])
                        out_b[r, pl.ds(_L * j, _L)] = v

                pltpu.async_copy(out_b, out_hbm.at[rows(g)], sout[b])

                @pl.when(g + _NBUF < nchunk)
                def _issue_in():
                    pltpu.async_copy(x_hbm.at[rows(g + _NBUF)], in_b, sin[b])
            return 0

        lax.fori_loop(0, ngroups, group_body, 0)

        for b in range(_NBUF):
            g_last = (ngroups - 1) * _NBUF + b
            pltpu.make_async_copy(
                out_bufs[b], out_hbm.at[rows(g_last)], sout[b]
            ).wait()

    return k(x_rows, idx)


def kernel(x, indexes):
    b, s, d = x.shape
    x_rows = x.reshape(b * s, d)
    idx = indexes.astype(jnp.int32)
    out = _sc_permute(x_rows, idx)
    return out.reshape(b, s, d)
